# baseline (speedup 1.0000x reference)
import functools

import jax
import jax.numpy as jnp
from jax import lax
from jax.experimental import pallas as pl
from jax.experimental.pallas import tpu as pltpu
from jax.experimental.pallas import tpu_sc as plsc

D = 128
NC, NS = 2, 16
NW = NC * NS
G_SUBS = (128, 72)
G_OFF = (0, 128)
CH = sum(G_SUBS)
NSLOT = 4


def _prep_body(x_ref, wm1_ref, bm1_ref, ww1_ref, bw1_ref,
               pam_ref, pbm_ref, paw_ref, pbw_ref):
    x = x_ref[...]
    f32 = jnp.float32
    pam_ref[...] = jnp.dot(x, wm1_ref[0:D, :], preferred_element_type=f32) + bm1_ref[...]
    pbm_ref[...] = jnp.dot(x, wm1_ref[D:2 * D, :], preferred_element_type=f32)
    paw_ref[...] = jnp.dot(x, ww1_ref[0:D, :], preferred_element_type=f32) + bw1_ref[...]
    pbw_ref[...] = jnp.dot(x, ww1_ref[D:2 * D, :], preferred_element_type=f32)


def _precompute_tables(x, wm1, bm1, ww1, bw1):
    n = x.shape[0]
    blk = n // 5
    tbl = jax.ShapeDtypeStruct((n, D), jnp.float32)
    row_spec = pl.BlockSpec((blk, D), lambda i: (i, 0))
    full = pl.BlockSpec((2 * D, D), lambda i: (0, 0))
    bias = pl.BlockSpec((1, D), lambda i: (0, 0))
    return pl.pallas_call(
        _prep_body,
        grid=(5,),
        in_specs=[row_spec, full, bias, full, bias],
        out_specs=(row_spec, row_spec, row_spec, row_spec),
        out_shape=(tbl, tbl, tbl, tbl),
    )(x, wm1[: 2 * D], bm1.reshape(1, D), ww1[: 2 * D], bw1.reshape(1, D))


def _gather_stream(table, idx_hbm, out_hbm, idx_v, slots, gsems, wsems,
                   base, n_edges):
    n = n_edges // CH
    pltpu.sync_copy(idx_hbm.at[pl.ds(base, n_edges)], idx_v.at[pl.ds(0, n_edges)])

    def issue(jj, p):
        for off, sub in zip(G_OFF, G_SUBS):
            pltpu.async_copy(
                table.at[idx_v.at[pl.ds(jj * CH + off, sub)]],
                slots[p].at[pl.ds(off, sub)],
                gsems[p],
            )

    def wait_gathers(p):
        for off, sub in zip(G_OFF, G_SUBS):
            pltpu.make_async_copy(table.at[pl.ds(0, sub)],
                                  slots[p].at[pl.ds(off, sub)], gsems[p]).wait()

    def wait_write(p):
        pltpu.make_async_copy(slots[p], out_hbm.at[pl.ds(base, CH)], wsems[p]).wait()

    issue(0, 0)
    issue(1, 1)

    def step(i, carry):
        for p in range(NSLOT):
            @pl.when(i % NSLOT == p)
            def _():
                wait_gathers(p)

                @pl.when(i >= 2)
                def _():
                    wait_write((p + 2) % NSLOT)

                @pl.when(i + 2 < n)
                def _():
                    issue(i + 2, (p + 2) % NSLOT)
                pltpu.async_copy(slots[p], out_hbm.at[pl.ds(base + i * CH, CH)],
                                 wsems[p])
        return carry

    lax.fori_loop(0, n, step, 0)
    wait_write((n - 1) % NSLOT)
    wait_write((n - 2) % NSLOT)


def _sc_body(ta, tb, s_idx, r_idx, ga, gb, idx_v,
             s0, s1, s2, s3, g0, g1, g2, g3, w0, w1, w2, w3):
    wid = lax.axis_index("s") * NC + lax.axis_index("c")
    ne = s_idx.shape[0] // NW
    slots = (s0, s1, s2, s3)
    gsems = (g0, g1, g2, g3)
    wsems = (w0, w1, w2, w3)
    _gather_stream(ta, s_idx, ga, idx_v, slots, gsems, wsems, wid * ne, ne)
    _gather_stream(tb, r_idx, gb, idx_v, slots, gsems, wsems, wid * ne, ne)


def _sc_gather(ta, tb, eidx):
    ne = eidx.shape[1]
    out = (jax.ShapeDtypeStruct((ne, D), jnp.float32),
           jax.ShapeDtypeStruct((ne, D), jnp.float32))
    k = pl.kernel(
        _sc_body,
        out_type=out,
        mesh=plsc.VectorSubcoreMesh(core_axis_name="c", subcore_axis_name="s",
                                    num_cores=NC, num_subcores=NS),
        scratch_types=[
            pltpu.VMEM((ne // NW,), jnp.int32),
            pltpu.VMEM((CH, D), jnp.float32),
            pltpu.VMEM((CH, D), jnp.float32),
            pltpu.VMEM((CH, D), jnp.float32),
            pltpu.VMEM((CH, D), jnp.float32),
            pltpu.SemaphoreType.DMA,
            pltpu.SemaphoreType.DMA,
            pltpu.SemaphoreType.DMA,
            pltpu.SemaphoreType.DMA,
            pltpu.SemaphoreType.DMA,
            pltpu.SemaphoreType.DMA,
            pltpu.SemaphoreType.DMA,
            pltpu.SemaphoreType.DMA,
        ],
    )
    return k(ta, tb, eidx[0], eidx[1])


def _mlp_body(ga_ref, gb_ref, e_ref, w1c_ref, w2_ref, b2_ref, out_ref):
    e = e_ref[...]
    bf = jnp.bfloat16
    h = (ga_ref[...] + gb_ref[...]
         + jnp.dot(e.astype(bf), w1c_ref[...].astype(bf),
                   preferred_element_type=jnp.float32))
    h = jnp.maximum(h, 0.0)
    out_ref[...] = e + jnp.dot(h.astype(bf), w2_ref[...].astype(bf),
                               preferred_element_type=jnp.float32) + b2_ref[...]


def _edge_mlp(ga, gb, e, w1c, w2, b2, blk):
    n = e.shape[0]
    gspec = pl.BlockSpec((blk, D), lambda i: (i, 0))
    wspec = pl.BlockSpec((D, D), lambda i: (0, 0))
    bias = pl.BlockSpec((1, D), lambda i: (0, 0))
    return pl.pallas_call(
        _mlp_body,
        grid=(n // blk,),
        in_specs=[gspec, gspec, gspec, wspec, wspec, bias],
        out_specs=gspec,
        out_shape=jax.ShapeDtypeStruct((n, D), jnp.float32),
        compiler_params=pltpu.CompilerParams(
            dimension_semantics=("arbitrary",)),
    )(ga, gb, e, w1c, w2, b2.reshape(1, D))


def kernel(x, mesh_edge_index, mesh_edge_attr, world_edge_index, world_edge_attr,
           Wm1, bm1, Wm2, bm2, Ww1, bw1, Ww2, bw2):
    pam, pbm, paw, pbw = _precompute_tables(x, Wm1, bm1, Ww1, bw1)
    gam, gbm = _sc_gather(pam, pbm, mesh_edge_index)
    gaw, gbw = _sc_gather(paw, pbw, world_edge_index)
    mesh_out = _edge_mlp(gam, gbm, mesh_edge_attr, Wm1[2 * D:], Wm2, bm2, 4000)
    world_out = _edge_mlp(gaw, gbw, world_edge_attr, Ww1[2 * D:], Ww2, bw2, 4000)
    return (mesh_out, world_out)
